# baseline (speedup 1.0000x reference)
import functools

import jax
import jax.numpy as jnp
from jax import lax
from jax.experimental import pallas as pl
from jax.experimental.pallas import tpu as pltpu
from jax.experimental.pallas import tpu_sc as plsc

MAX_REL = 32
HIDDEN = 128
NC, NS, L = 2, 16, 16
NW = NC * NS
K = 128
NBUF = 4


def _sc_lookup(n_total: int):
    c_per_w = n_total // NW
    nb = c_per_w // K
    mesh = plsc.VectorSubcoreMesh(core_axis_name="c", subcore_axis_name="s")

    @functools.partial(
        pl.kernel,
        out_type=jax.ShapeDtypeStruct((n_total, HIDDEN), jnp.float32),
        mesh=mesh,
        scratch_types=[
            pltpu.VMEM((NBUF, 1, HIDDEN), jnp.int32),
            pltpu.VMEM((NBUF, 1, HIDDEN), jnp.int32),
            pltpu.VMEM((NBUF * K, HIDDEN), jnp.float32),
            pltpu.VMEM_SHARED((66, HIDDEN), jnp.float32),
            pltpu.SemaphoreType.DMA,
        ] + [pltpu.SemaphoreType.DMA] * NBUF
          + [pltpu.SemaphoreType.DMA] * (NBUF // 2),
    )
    def kfn(pk_hbm, table_hbm, out_hbm, pk_v, d_v, rows_v, table_v,
            sem_i, *sems):
        sem_g, sem_s = sems[:NBUF], sems[NBUF:]
        wid = lax.axis_index("s") * NC + lax.axis_index("c")
        row0 = wid * (c_per_w // HIDDEN)

        @pl.when(lax.axis_index("s") == 0)
        def _stage_table():
            pltpu.sync_copy(table_hbm, table_v)

        plsc.subcore_barrier()

        def stage_idx(cb, ring):
            pltpu.async_copy(pk_hbm.at[pl.ds(row0 + cb, 1)], pk_v.at[ring],
                             sem_i)

        def wait_idx(cb, ring):
            pltpu.make_async_copy(pk_hbm.at[pl.ds(row0 + cb, 1)],
                                  pk_v.at[ring], sem_i).wait()

        def gather_desc(ring):
            return pltpu.make_async_copy(
                table_v.at[d_v.at[ring, 0]],
                rows_v.at[pl.ds(ring * K, K)], sem_g[ring])

        def scatter_desc(cb_even, pair):
            r = (row0 + cb_even) * HIDDEN
            return pltpu.make_async_copy(
                rows_v.at[pl.ds(pair * 2 * K, 2 * K)],
                out_hbm.at[pl.ds(r, 2 * K)], sem_s[pair])

        stage_idx(0, 0)

        @pl.loop(0, nb, step=NBUF)
        def _group(v):
            for ring in range(NBUF):
                cb = v + ring
                pair = ring // 2
                wait_idx(cb, ring)

                @pl.when(cb + 1 < nb)
                def _prefetch():
                    stage_idx(cb + 1, (ring + 1) % NBUF)

                for i in range(HIDDEN // L):
                    pk = pk_v[ring, 0, pl.ds(i * L, L)]
                    off = pk & 0xFFF
                    m = pk >> 12
                    dc = jnp.clip(off + MAX_REL, 0, 2 * MAX_REL)
                    d_v[ring, 0, pl.ds(i * L, L)] = (
                        dc * m + (1 - m) * (2 * MAX_REL + 1))

                if ring % 2 == 0:
                    @pl.when(cb >= NBUF)
                    def _drain_scatter():
                        scatter_desc(cb - NBUF, pair).wait()

                gather_desc(ring).start()

                if ring % 2 == 1:
                    gather_desc(ring - 1).wait()
                    gather_desc(ring).wait()
                    scatter_desc(cb - 1, pair).start()

        scatter_desc(nb - NBUF, 0).wait()
        scatter_desc(nb - 2, 1).wait()

    return kfn


@jax.jit
def kernel(offset, mask, emb_table):
    b, s = offset.shape
    n = b * s
    packed = (offset.astype(jnp.int32)
              | (mask.astype(jnp.int32) << 12)).reshape(n // HIDDEN, HIDDEN)
    out = _sc_lookup(n)(packed, emb_table)
    return out.reshape(b, s, HIDDEN)


# speedup vs baseline: 1.0698x; 1.0698x over previous
import functools

import jax
import jax.numpy as jnp
from jax import lax
from jax.experimental import pallas as pl
from jax.experimental.pallas import tpu as pltpu
from jax.experimental.pallas import tpu_sc as plsc

MAX_REL = 32
HIDDEN = 128
NC, NS, L = 2, 16, 16
NW = NC * NS
K = 128
NBUF = 5


def _sc_lookup(n_total: int):
    c_per_w = n_total // NW
    nb = c_per_w // K
    mesh = plsc.VectorSubcoreMesh(core_axis_name="c", subcore_axis_name="s")

    @functools.partial(
        pl.kernel,
        out_type=jax.ShapeDtypeStruct((n_total, HIDDEN), jnp.float32),
        mesh=mesh,
        scratch_types=[
            pltpu.VMEM((NBUF, 1, HIDDEN), jnp.int32),
            pltpu.VMEM((NBUF, 1, HIDDEN), jnp.int32),
            pltpu.VMEM((NBUF, K, HIDDEN), jnp.float32),
            pltpu.VMEM_SHARED((66, HIDDEN), jnp.float32),
            pltpu.SemaphoreType.DMA,
        ] + [pltpu.SemaphoreType.DMA] * NBUF
          + [pltpu.SemaphoreType.DMA] * NBUF,
    )
    def kfn(pk_hbm, table_hbm, out_hbm, pk_v, d_v, rows_v, table_v,
            sem_i, *sems):
        sem_g, sem_s = sems[:NBUF], sems[NBUF:]
        wid = lax.axis_index("s") * NC + lax.axis_index("c")
        row0 = wid * (c_per_w // HIDDEN)

        @pl.when(lax.axis_index("s") == 0)
        def _stage_table():
            pltpu.sync_copy(table_hbm, table_v)

        plsc.subcore_barrier()

        def stage_idx(cb, ring):
            pltpu.async_copy(pk_hbm.at[pl.ds(row0 + cb, 1)], pk_v.at[ring],
                             sem_i)

        def wait_idx(cb, ring):
            pltpu.make_async_copy(pk_hbm.at[pl.ds(row0 + cb, 1)],
                                  pk_v.at[ring], sem_i).wait()

        def gather_desc(ring):
            return pltpu.make_async_copy(
                table_v.at[d_v.at[ring, 0]], rows_v.at[ring], sem_g[ring])

        def scatter_desc(cb, ring):
            r = (row0 + cb) * HIDDEN
            return pltpu.make_async_copy(
                rows_v.at[ring], out_hbm.at[pl.ds(r, K)], sem_s[ring])

        stage_idx(0, 0)

        @pl.loop(0, nb, step=NBUF)
        def _group(v):
            for ring in range(NBUF):
                cb = v + ring
                wait_idx(cb, ring)

                @pl.when(cb + 1 < nb)
                def _prefetch():
                    stage_idx(cb + 1, (ring + 1) % NBUF)

                for i in range(HIDDEN // L):
                    pk = pk_v[ring, 0, pl.ds(i * L, L)]
                    off = pk & 0xFFF
                    m = pk >> 12
                    dc = jnp.clip(off + MAX_REL, 0, 2 * MAX_REL)
                    d_v[ring, 0, pl.ds(i * L, L)] = (
                        dc * m + (1 - m) * (2 * MAX_REL + 1))

                @pl.when(cb >= NBUF)
                def _drain_scatter():
                    scatter_desc(cb - NBUF, ring).wait()

                gather_desc(ring).start()

                @pl.when(cb >= 1)
                def _emit_prev():
                    gather_desc((ring + NBUF - 1) % NBUF).wait()
                    scatter_desc(cb - 1, (ring + NBUF - 1) % NBUF).start()

        last = nb - 1
        gather_desc(last % NBUF).wait()
        scatter_desc(last, last % NBUF).start()
        for t in range(NBUF):
            cb = nb - NBUF + t
            scatter_desc(cb, cb % NBUF).wait()

    return kfn


@jax.jit
def kernel(offset, mask, emb_table):
    b, s = offset.shape
    n = b * s
    packed = (offset.astype(jnp.int32)
              | (mask.astype(jnp.int32) << 12)).reshape(n // HIDDEN, HIDDEN)
    out = _sc_lookup(n)(packed, emb_table)
    return out.reshape(b, s, HIDDEN)
